# baseline (speedup 1.0000x reference)
import functools

import numpy as np

import jax
import jax.numpy as jnp
from jax.experimental import pallas as pl
from jax.experimental.pallas import tpu as pltpu

_EPS = 1e-8
_ROW_TILE = 1024


def _make_w(n: int) -> np.ndarray:
    nm1 = n - 1
    k = np.arange(n)[:, None]
    i = np.arange(n)[None, :]
    w = np.where(k <= i, 1.0, 0.0)
    w = np.where((k == 0) | (k == i + 1), 0.5, w)
    w = np.where(i >= nm1, 0.0, w)
    return w.astype(jnp.bfloat16)


def _make_wtot(n: int) -> np.ndarray:
    k = np.arange(n)[:, None]
    wt = np.where((k == 0) | (k == n - 1), 0.5, 1.0)
    return np.pad(wt, ((0, 0), (0, 127))).astype(np.float32)


def _w1_kernel(d_ref, r_ref, w_ref, wtot_ref, out_ref, *, eps, n, c):
    ad = jnp.abs(d_ref[...])
    ar = jnp.abs(r_ref[...])
    wtot = wtot_ref[...]
    tot_d = jnp.dot(ad, wtot, preferred_element_type=jnp.float32)[:, :1]
    tot_r = jnp.dot(ar, wtot, preferred_element_type=jnp.float32)[:, :1]
    inv_d = pl.reciprocal(eps + tot_d, approx=False)
    inv_r = pl.reciprocal(eps + tot_r, approx=False)
    s = (ad * inv_d - ar * inv_r).astype(jnp.bfloat16)
    diff = jnp.dot(s, w_ref[...], preferred_element_type=jnp.float32)
    per_row = jnp.sum(diff * diff, axis=1)
    per_batch = jnp.sum(per_row.reshape(-1, c), axis=1, keepdims=True)
    out_ref[...] = per_batch * (1.0 / (c * (n - 1)))


def kernel(data, ref_data):
    B, C, N = data.shape
    rows = B * C
    d = data.reshape(rows, N)
    r = ref_data.reshape(rows, N)
    tile = min(_ROW_TILE, rows)
    w = _make_w(N)
    wtot = _make_wtot(N)

    per_batch = pl.pallas_call(
        functools.partial(_w1_kernel, eps=_EPS, n=N, c=C),
        out_shape=jax.ShapeDtypeStruct((rows // C, 1), jnp.float32),
        grid=(pl.cdiv(rows, tile),),
        in_specs=[
            pl.BlockSpec((tile, N), lambda i: (i, 0)),
            pl.BlockSpec((tile, N), lambda i: (i, 0)),
            pl.BlockSpec((N, N), lambda i: (0, 0), pipeline_mode=pl.Buffered(1)),
            pl.BlockSpec((N, 128), lambda i: (0, 0), pipeline_mode=pl.Buffered(1)),
        ],
        out_specs=pl.BlockSpec((tile // C, 1), lambda i: (i, 0)),
        compiler_params=pltpu.CompilerParams(
            dimension_semantics=("parallel",),
            vmem_limit_bytes=48 * 1024 * 1024),
        cost_estimate=pl.CostEstimate(
            flops=2 * rows * N * N + 8 * rows * N,
            transcendentals=0,
            bytes_accessed=(d.size + r.size) * d.dtype.itemsize + 4 * rows // C),
    )(d, r, w, wtot)

    return per_batch[:, 0]


# speedup vs baseline: 1.0468x; 1.0468x over previous
import functools

import numpy as np

import jax
import jax.numpy as jnp
from jax.experimental import pallas as pl
from jax.experimental.pallas import tpu as pltpu

_EPS = 1e-8
_ROW_TILE = 2048


def _make_w(n: int) -> np.ndarray:
    nm1 = n - 1
    k = np.arange(n)[:, None]
    i = np.arange(n)[None, :]
    w = np.where(k <= i, 1.0, 0.0)
    w = np.where((k == 0) | (k == i + 1), 0.5, w)
    w = np.where(i >= nm1, 0.0, w)
    return w.astype(jnp.bfloat16)


def _make_wtot(n: int) -> np.ndarray:
    k = np.arange(n)[:, None]
    wt = np.where((k == 0) | (k == n - 1), 0.5, 1.0)
    return np.pad(wt, ((0, 0), (0, 127))).astype(np.float32)


def _w1_kernel(d_ref, r_ref, w_ref, wtot_ref, out_ref, *, eps, n, c):
    ad = jnp.abs(d_ref[...])
    ar = jnp.abs(r_ref[...])
    wtot = wtot_ref[...]
    tot_d = jnp.dot(ad, wtot, preferred_element_type=jnp.float32)[:, :1]
    tot_r = jnp.dot(ar, wtot, preferred_element_type=jnp.float32)[:, :1]
    inv_d = pl.reciprocal(eps + tot_d, approx=False)
    inv_r = pl.reciprocal(eps + tot_r, approx=False)
    s = (ad * inv_d - ar * inv_r).astype(jnp.bfloat16)
    diff = jnp.dot(s, w_ref[...], preferred_element_type=jnp.float32)
    per_row = jnp.sum(diff * diff, axis=1)
    per_batch = jnp.sum(per_row.reshape(-1, c), axis=1, keepdims=True)
    out_ref[...] = per_batch * (1.0 / (c * (n - 1)))


def kernel(data, ref_data):
    B, C, N = data.shape
    rows = B * C
    d = data.reshape(rows, N)
    r = ref_data.reshape(rows, N)
    tile = min(_ROW_TILE, rows)
    w = _make_w(N)
    wtot = _make_wtot(N)

    per_batch = pl.pallas_call(
        functools.partial(_w1_kernel, eps=_EPS, n=N, c=C),
        out_shape=jax.ShapeDtypeStruct((rows // C, 1), jnp.float32),
        grid=(pl.cdiv(rows, tile),),
        in_specs=[
            pl.BlockSpec((tile, N), lambda i: (i, 0)),
            pl.BlockSpec((tile, N), lambda i: (i, 0)),
            pl.BlockSpec((N, N), lambda i: (0, 0), pipeline_mode=pl.Buffered(1)),
            pl.BlockSpec((N, 128), lambda i: (0, 0), pipeline_mode=pl.Buffered(1)),
        ],
        out_specs=pl.BlockSpec((tile // C, 1), lambda i: (i, 0)),
        compiler_params=pltpu.CompilerParams(
            dimension_semantics=("parallel",),
            vmem_limit_bytes=48 * 1024 * 1024),
        cost_estimate=pl.CostEstimate(
            flops=2 * rows * N * N + 8 * rows * N,
            transcendentals=0,
            bytes_accessed=(d.size + r.size) * d.dtype.itemsize + 4 * rows // C),
    )(d, r, w, wtot)

    return per_batch[:, 0]
